# baseline (speedup 1.0000x reference)
import functools

import jax
import jax.numpy as jnp
from jax import lax
from jax.experimental import pallas as pl
from jax.experimental.pallas import tpu as pltpu
from jax.experimental.pallas import tpu_sc as plsc

VOCAB = 1000000
EMBED = 64
B = 16384
CTX = 20
T = 5

NC, NS, L = 2, 16, 16
NW = NC * NS
ROWS_PER_W = B // NW
C = 32
NCHUNK = ROWS_PER_W // C
NSEG = EMBED // L
ROW_W = 128
CTX_IDX_MINOR = 128
TGT_IDX_MINOR = 80

HALF = 524288
VB = 16384
CONV_GRID = HALF // VB


def _transpose_kernel(in_ref, out_ref):
    p = pl.program_id(1)
    t = jnp.transpose(in_ref[:, :])

    @pl.when(p == 0)
    def _():
        out_ref[:, 0:EMBED] = t

    @pl.when(p == 1)
    def _():
        out_ref[:, EMBED:ROW_W] = t


def _convert(wt):
    return pl.pallas_call(
        _transpose_kernel,
        grid=(CONV_GRID, 2),
        in_specs=[pl.BlockSpec(
            (EMBED, VB),
            lambda g, p: (0, jnp.minimum(g + p * CONV_GRID, VOCAB // VB)))],
        out_specs=pl.BlockSpec((VB, ROW_W), lambda g, p: (g, 0)),
        out_shape=jax.ShapeDtypeStruct((HALF, ROW_W), jnp.float32),
    )(wt)


def _sc_logits_kernel():
    mesh = plsc.VectorSubcoreMesh(
        core_axis_name="c", subcore_axis_name="s", num_cores=NC, num_subcores=NS
    )

    @functools.partial(
        pl.kernel,
        out_type=jax.ShapeDtypeStruct((B * T,), jnp.float32),
        mesh=mesh,
        scratch_types=[
            pltpu.VMEM((C * CTX // CTX_IDX_MINOR, CTX_IDX_MINOR), jnp.int32),
            pltpu.VMEM((C * T // TGT_IDX_MINOR, TGT_IDX_MINOR), jnp.int32),
            pltpu.VMEM((C * CTX // CTX_IDX_MINOR, CTX_IDX_MINOR), jnp.int32),
            pltpu.VMEM((C * T // TGT_IDX_MINOR, TGT_IDX_MINOR), jnp.int32),
            pltpu.VMEM((C * CTX,), jnp.int32),
            pltpu.VMEM((C * 8 + 8,), jnp.int32),
            pltpu.VMEM((C * CTX, ROW_W), jnp.float32),
            pltpu.VMEM((C * T, ROW_W), jnp.float32),
            pltpu.VMEM((C * T * L,), jnp.float32),
            pltpu.VMEM((ROWS_PER_W * T,), jnp.float32),
            pltpu.SemaphoreType.DMA,
        ],
        compiler_params=pltpu.CompilerParams(needs_layout_passes=False,
                                             use_tc_tiling_on_sc=False),
    )
    def k(ctx_idx_hbm, tgt_idx_hbm, win_hbm, wout_hbm, out_hbm,
          idx_c, idx_t, idxp_c, idxp_t, hv_c, hv_t8,
          ctx_v, tgt_v, part_v, log_v, sem):
        wid = lax.axis_index("s") * NC + lax.axis_index("c")
        ctx_off0 = wid * (ROWS_PER_W * CTX)
        tgt_off0 = wid * (ROWS_PER_W * T)
        lane = lax.iota(jnp.int32, L)

        def chunk_body(g, carry):
            c_off = pl.multiple_of(ctx_off0 + g * (C * CTX), 8)
            t_off = pl.multiple_of(tgt_off0 + g * (C * T), 8)
            for i in range(C * CTX // CTX_IDX_MINOR):
                pltpu.sync_copy(
                    ctx_idx_hbm.at[pl.ds(c_off + i * CTX_IDX_MINOR,
                                         CTX_IDX_MINOR)],
                    idx_c.at[i])
            for i in range(C * T // TGT_IDX_MINOR):
                pltpu.sync_copy(
                    tgt_idx_hbm.at[pl.ds(t_off + i * TGT_IDX_MINOR,
                                         TGT_IDX_MINOR)],
                    idx_t.at[i])
            for i in range(C * CTX // CTX_IDX_MINOR):
                for m in range(CTX_IDX_MINOR // L):
                    vv = idx_c[i, pl.ds(m * L, L)]
                    idxp_c[i, pl.ds(m * L, L)] = vv & (HALF - 1)
                    hv_c[pl.ds(i * CTX_IDX_MINOR + m * L, L)] = (vv >> 13) & EMBED
            for i in range(C * T // TGT_IDX_MINOR):
                for m in range(TGT_IDX_MINOR // L):
                    vv = idx_t[i, pl.ds(m * L, L)]
                    idxp_t[i, pl.ds(m * L, L)] = vv & (HALF - 1)
                    tr_vec = (i * TGT_IDX_MINOR + m * L) + lane
                    pos = (tr_vec // T) * 8 + tr_vec % T
                    plsc.store_scatter(hv_t8, [pos], (vv >> 13) & EMBED)
            cps = []
            for i in range(C * CTX // CTX_IDX_MINOR):
                cps.append(pltpu.async_copy(
                    win_hbm.at[idxp_c.at[i]],
                    ctx_v.at[pl.ds(i * CTX_IDX_MINOR, CTX_IDX_MINOR)], sem))
            for i in range(C * T // TGT_IDX_MINOR):
                cps.append(pltpu.async_copy(
                    wout_hbm.at[idxp_t.at[i]],
                    tgt_v.at[pl.ds(i * TGT_IDX_MINOR, TGT_IDX_MINOR)], sem))
            for cp in cps:
                cp.wait()

            def row_body(r, rc):
                base_c = r * CTX
                ro = pl.multiple_of(r * CTX, 4)
                hv1 = hv_c[pl.ds(ro, L)]
                hv2 = hv_c[pl.ds(ro + 4, L)]
                htv = hv_t8[pl.ds(pl.multiple_of(r * 8, 8), L)]
                acc = [jnp.zeros((L,), jnp.float32) for _ in range(NSEG)]
                for c in range(CTX):
                    h = hv1[c] if c < L else hv2[c - 4]
                    for j in range(NSEG):
                        acc[j] = acc[j] + ctx_v[base_c + c, pl.ds(h + j * L, L)]
                for t in range(T):
                    tr = r * T + t
                    ht = htv[t]
                    s = acc[0] * tgt_v[tr, pl.ds(ht, L)]
                    for j in range(1, NSEG):
                        s = s + acc[j] * tgt_v[tr, pl.ds(ht + j * L, L)]
                    po = pl.multiple_of(tr * L, 16)
                    part_v[pl.ds(po, L)] = s
                return rc

            lax.fori_loop(0, C, row_body, 0)

            for m in range(C * T // L):
                idx0 = lane * L + (m * L * L)
                red = plsc.load_gather(part_v, [idx0])
                for kk in range(1, L):
                    red = red + plsc.load_gather(part_v, [idx0 + kk])
                lo = pl.multiple_of(g * (C * T) + m * L, 16)
                log_v[pl.ds(lo, L)] = red * (1.0 / CTX)
            return carry

        lax.fori_loop(0, NCHUNK, chunk_body, 0)
        pltpu.sync_copy(
            log_v,
            out_hbm.at[pl.ds(wid * (ROWS_PER_W * T), ROWS_PER_W * T)])

    return k


def _bce_kernel(logits_ref, labels_ref, out_ref):
    l = logits_ref[:, :]
    y = labels_ref[:, :]
    bce = jnp.maximum(l, 0.0) - l * y + jnp.log(1.0 + jnp.exp(-jnp.abs(l)))
    out_ref[0, 0] = jnp.sum(bce) * (1.0 / (B * T))


@jax.jit
def kernel(contexts, targets, labels, W_in, W_out):
    ctx_idx = contexts.astype(jnp.int32).reshape(B * CTX)
    tgt_idx = targets.astype(jnp.int32).reshape(B * T)
    win_c = _convert(W_in.T)
    wout_c = _convert(W_out.T)
    logits = _sc_logits_kernel()(ctx_idx, tgt_idx, win_c, wout_c)

    labels_f = labels.astype(jnp.float32).reshape(B * T)
    loss2d = pl.pallas_call(
        _bce_kernel,
        out_shape=jax.ShapeDtypeStruct((1, 1), jnp.float32),
        in_specs=[pl.BlockSpec(memory_space=pltpu.VMEM),
                  pl.BlockSpec(memory_space=pltpu.VMEM)],
        out_specs=pl.BlockSpec(memory_space=pltpu.SMEM),
    )(logits.reshape(B * T // 128, 128),
      labels_f.reshape(B * T // 128, 128))
    return loss2d[0, 0]


# speedup vs baseline: 1.3777x; 1.3777x over previous
import functools

import jax
import jax.numpy as jnp
from jax import lax
from jax.experimental import pallas as pl
from jax.experimental.pallas import tpu as pltpu
from jax.experimental.pallas import tpu_sc as plsc

VOCAB = 1000000
EMBED = 64
B = 16384
CTX = 20
T = 5

NC, NS, L = 2, 16, 16
NW = NC * NS
ROWS_PER_W = B // NW
C = 32
NCHUNK = ROWS_PER_W // C
NSEG = EMBED // L
ROW_W = 128
CTX_IDX_MINOR = 128
TGT_IDX_MINOR = 80

HALF = 524288
VB = 16384
CONV_GRID = HALF // VB


def _transpose_kernel(lo_ref, hi_ref, out_ref):
    out_ref[:, 0:EMBED] = jnp.transpose(lo_ref[:, :])
    out_ref[:, EMBED:ROW_W] = jnp.transpose(hi_ref[:, :])


def _convert(wt):
    return pl.pallas_call(
        _transpose_kernel,
        grid=(CONV_GRID,),
        in_specs=[pl.BlockSpec((EMBED, VB), lambda g: (0, g)),
                  pl.BlockSpec((EMBED, VB),
                               lambda g: (0, jnp.minimum(g + CONV_GRID,
                                                         VOCAB // VB)))],
        out_specs=pl.BlockSpec((VB, ROW_W), lambda g: (g, 0)),
        out_shape=jax.ShapeDtypeStruct((HALF, ROW_W), jnp.float32),
    )(wt, wt)


_MESH = None


def _mesh():
    global _MESH
    if _MESH is None:
        _MESH = plsc.VectorSubcoreMesh(
            core_axis_name="c", subcore_axis_name="s",
            num_cores=NC, num_subcores=NS)
    return _MESH


def _sc_ctxsum_kernel():
    @functools.partial(
        pl.kernel,
        out_type=jax.ShapeDtypeStruct((B * EMBED,), jnp.float32),
        mesh=_mesh(),
        scratch_types=[
            pltpu.VMEM((C * CTX // CTX_IDX_MINOR, CTX_IDX_MINOR), jnp.int32),
            pltpu.VMEM((C * CTX // CTX_IDX_MINOR, CTX_IDX_MINOR), jnp.int32),
            pltpu.VMEM((C * CTX,), jnp.int32),
            pltpu.VMEM((C * CTX, ROW_W), jnp.float32),
            pltpu.VMEM((ROWS_PER_W * EMBED,), jnp.float32),
            pltpu.SemaphoreType.DMA,
        ],
        compiler_params=pltpu.CompilerParams(needs_layout_passes=False,
                                             use_tc_tiling_on_sc=False),
    )
    def k(ctx_idx_hbm, win_hbm, out_hbm, idx_c, idxp_c, hv_c, ctx_v,
          csum_v, sem):
        wid = lax.axis_index("s") * NC + lax.axis_index("c")
        ctx_off0 = wid * (ROWS_PER_W * CTX)

        def chunk_body(g, carry):
            c_off = pl.multiple_of(ctx_off0 + g * (C * CTX), 8)
            for i in range(C * CTX // CTX_IDX_MINOR):
                pltpu.sync_copy(
                    ctx_idx_hbm.at[pl.ds(c_off + i * CTX_IDX_MINOR,
                                         CTX_IDX_MINOR)],
                    idx_c.at[i])
            for i in range(C * CTX // CTX_IDX_MINOR):
                for m in range(CTX_IDX_MINOR // L):
                    vv = idx_c[i, pl.ds(m * L, L)]
                    idxp_c[i, pl.ds(m * L, L)] = vv & (HALF - 1)
                    hv_c[pl.ds(i * CTX_IDX_MINOR + m * L, L)] = (vv >> 13) & EMBED
            cps = []
            for i in range(C * CTX // CTX_IDX_MINOR):
                cps.append(pltpu.async_copy(
                    win_hbm.at[idxp_c.at[i]],
                    ctx_v.at[pl.ds(i * CTX_IDX_MINOR, CTX_IDX_MINOR)], sem))
            for cp in cps:
                cp.wait()

            def row_body(r, rc):
                base_c = r * CTX
                ro = pl.multiple_of(r * CTX, 4)
                hv1 = hv_c[pl.ds(ro, L)]
                hv2 = hv_c[pl.ds(ro + 4, L)]
                acc = [jnp.zeros((L,), jnp.float32) for _ in range(NSEG)]
                for c in range(CTX):
                    h = hv1[c] if c < L else hv2[c - 4]
                    for j in range(NSEG):
                        acc[j] = acc[j] + ctx_v[base_c + c, pl.ds(h + j * L, L)]
                so = pl.multiple_of((g * C + r) * EMBED, 16)
                for j in range(NSEG):
                    csum_v[pl.ds(so + j * L, L)] = acc[j]
                return rc

            lax.fori_loop(0, C, row_body, 0)
            return carry

        lax.fori_loop(0, NCHUNK, chunk_body, 0)
        pltpu.sync_copy(
            csum_v,
            out_hbm.at[pl.ds(wid * (ROWS_PER_W * EMBED), ROWS_PER_W * EMBED)])

    return k


def _sc_logits_kernel():
    @functools.partial(
        pl.kernel,
        out_type=jax.ShapeDtypeStruct((B * T,), jnp.float32),
        mesh=_mesh(),
        scratch_types=[
            pltpu.VMEM((C * T // TGT_IDX_MINOR, TGT_IDX_MINOR), jnp.int32),
            pltpu.VMEM((C * T // TGT_IDX_MINOR, TGT_IDX_MINOR), jnp.int32),
            pltpu.VMEM((C * 8 + 8,), jnp.int32),
            pltpu.VMEM((C * T, ROW_W), jnp.float32),
            pltpu.VMEM((C * EMBED,), jnp.float32),
            pltpu.VMEM((C * T * L,), jnp.float32),
            pltpu.VMEM((ROWS_PER_W * T,), jnp.float32),
            pltpu.SemaphoreType.DMA,
        ],
        compiler_params=pltpu.CompilerParams(needs_layout_passes=False,
                                             use_tc_tiling_on_sc=False),
    )
    def k(tgt_idx_hbm, wout_hbm, csum_hbm, out_hbm,
          idx_t, idxp_t, hv_t8, tgt_v, csum_v, part_v, log_v, sem):
        wid = lax.axis_index("s") * NC + lax.axis_index("c")
        tgt_off0 = wid * (ROWS_PER_W * T)
        csum_off0 = wid * (ROWS_PER_W * EMBED)
        lane = lax.iota(jnp.int32, L)

        def chunk_body(g, carry):
            t_off = pl.multiple_of(tgt_off0 + g * (C * T), 8)
            for i in range(C * T // TGT_IDX_MINOR):
                pltpu.sync_copy(
                    tgt_idx_hbm.at[pl.ds(t_off + i * TGT_IDX_MINOR,
                                         TGT_IDX_MINOR)],
                    idx_t.at[i])
            pltpu.sync_copy(
                csum_hbm.at[pl.ds(
                    pl.multiple_of(csum_off0 + g * (C * EMBED), 8),
                    C * EMBED)],
                csum_v)
            for i in range(C * T // TGT_IDX_MINOR):
                for m in range(TGT_IDX_MINOR // L):
                    vv = idx_t[i, pl.ds(m * L, L)]
                    idxp_t[i, pl.ds(m * L, L)] = vv & (HALF - 1)
                    tr_vec = (i * TGT_IDX_MINOR + m * L) + lane
                    pos = (tr_vec // T) * 8 + tr_vec % T
                    plsc.store_scatter(hv_t8, [pos], (vv >> 13) & EMBED)
            cps = []
            for i in range(C * T // TGT_IDX_MINOR):
                cps.append(pltpu.async_copy(
                    wout_hbm.at[idxp_t.at[i]],
                    tgt_v.at[pl.ds(i * TGT_IDX_MINOR, TGT_IDX_MINOR)], sem))
            for cp in cps:
                cp.wait()

            def row_body(r, rc):
                so = pl.multiple_of(r * EMBED, 16)
                acc = [csum_v[pl.ds(so + j * L, L)] for j in range(NSEG)]
                htv = hv_t8[pl.ds(pl.multiple_of(r * 8, 8), L)]
                for t in range(T):
                    tr = r * T + t
                    ht = htv[t]
                    s = acc[0] * tgt_v[tr, pl.ds(ht, L)]
                    for j in range(1, NSEG):
                        s = s + acc[j] * tgt_v[tr, pl.ds(ht + j * L, L)]
                    po = pl.multiple_of(tr * L, 16)
                    part_v[pl.ds(po, L)] = s
                return rc

            lax.fori_loop(0, C, row_body, 0)

            for m in range(C * T // L):
                idx0 = lane * L + (m * L * L)
                red = plsc.load_gather(part_v, [idx0])
                for kk in range(1, L):
                    red = red + plsc.load_gather(part_v, [idx0 + kk])
                lo = pl.multiple_of(g * (C * T) + m * L, 16)
                log_v[pl.ds(lo, L)] = red * (1.0 / CTX)
            return carry

        lax.fori_loop(0, NCHUNK, chunk_body, 0)
        pltpu.sync_copy(
            log_v,
            out_hbm.at[pl.ds(wid * (ROWS_PER_W * T), ROWS_PER_W * T)])

    return k


def _bce_kernel(logits_ref, labels_ref, out_ref):
    l = logits_ref[:, :]
    y = labels_ref[:, :]
    bce = jnp.maximum(l, 0.0) - l * y + jnp.log(1.0 + jnp.exp(-jnp.abs(l)))
    out_ref[0, 0] = jnp.sum(bce) * (1.0 / (B * T))


@jax.jit
def kernel(contexts, targets, labels, W_in, W_out):
    ctx_idx = contexts.astype(jnp.int32).reshape(B * CTX)
    tgt_idx = targets.astype(jnp.int32).reshape(B * T)
    win_c = _convert(W_in.T)
    csum = _sc_ctxsum_kernel()(ctx_idx, win_c)
    wout_c = _convert(W_out.T)
    logits = _sc_logits_kernel()(tgt_idx, wout_c, csum)

    labels_f = labels.astype(jnp.float32).reshape(B * T)
    loss2d = pl.pallas_call(
        _bce_kernel,
        out_shape=jax.ShapeDtypeStruct((1, 1), jnp.float32),
        in_specs=[pl.BlockSpec(memory_space=pltpu.VMEM),
                  pl.BlockSpec(memory_space=pltpu.VMEM)],
        out_specs=pl.BlockSpec(memory_space=pltpu.SMEM),
    )(logits.reshape(B * T // 128, 128),
      labels_f.reshape(B * T // 128, 128))
    return loss2d[0, 0]


# speedup vs baseline: 1.4017x; 1.0174x over previous
import functools

import jax
import jax.numpy as jnp
from jax import lax
from jax.experimental import pallas as pl
from jax.experimental.pallas import tpu as pltpu
from jax.experimental.pallas import tpu_sc as plsc

VOCAB = 1000000
EMBED = 64
B = 16384
CTX = 20
T = 5

NC, NS, L = 2, 16, 16
NW = NC * NS
ROWS_PER_W = B // NW
C = 32
NCHUNK = ROWS_PER_W // C
NSEG = EMBED // L
ROW_W = 128
CTX_IDX_MINOR = 128
TGT_IDX_MINOR = 80

VB = 32768
VBH = VB // 2
PAIR_M = VBH - 1
CONV_GRID = (VOCAB + VB - 1) // VB
CONV_ROWS = CONV_GRID * VBH


def _transpose_kernel(wt_ref, out_ref):
    t = jnp.transpose(wt_ref[:, :])
    out_ref[:, 0:EMBED] = t[0:VBH]
    out_ref[:, EMBED:ROW_W] = t[VBH:VB]


def _convert(wt):
    return pl.pallas_call(
        _transpose_kernel,
        grid=(CONV_GRID,),
        in_specs=[pl.BlockSpec((EMBED, VB), lambda g: (0, g))],
        out_specs=pl.BlockSpec((VBH, ROW_W), lambda g: (g, 0)),
        out_shape=jax.ShapeDtypeStruct((CONV_ROWS, ROW_W), jnp.float32),
    )(wt)


_MESH = None


def _mesh():
    global _MESH
    if _MESH is None:
        _MESH = plsc.VectorSubcoreMesh(
            core_axis_name="c", subcore_axis_name="s",
            num_cores=NC, num_subcores=NS)
    return _MESH


def _sc_ctxsum_kernel():
    @functools.partial(
        pl.kernel,
        out_type=jax.ShapeDtypeStruct((B * EMBED,), jnp.float32),
        mesh=_mesh(),
        scratch_types=[
            pltpu.VMEM((C * CTX // CTX_IDX_MINOR, CTX_IDX_MINOR), jnp.int32),
            pltpu.VMEM((C * CTX // CTX_IDX_MINOR, CTX_IDX_MINOR), jnp.int32),
            pltpu.VMEM((C * CTX,), jnp.int32),
            pltpu.VMEM((C * CTX, ROW_W), jnp.float32),
            pltpu.VMEM((ROWS_PER_W * EMBED,), jnp.float32),
            pltpu.SemaphoreType.DMA,
        ],
        compiler_params=pltpu.CompilerParams(needs_layout_passes=False,
                                             use_tc_tiling_on_sc=False),
    )
    def k(ctx_idx_hbm, win_hbm, out_hbm, idx_c, idxp_c, hv_c, ctx_v,
          csum_v, sem):
        wid = lax.axis_index("s") * NC + lax.axis_index("c")
        ctx_off0 = wid * (ROWS_PER_W * CTX)

        def chunk_body(g, carry):
            c_off = pl.multiple_of(ctx_off0 + g * (C * CTX), 8)
            for i in range(C * CTX // CTX_IDX_MINOR):
                pltpu.sync_copy(
                    ctx_idx_hbm.at[pl.ds(c_off + i * CTX_IDX_MINOR,
                                         CTX_IDX_MINOR)],
                    idx_c.at[i])
            for i in range(C * CTX // CTX_IDX_MINOR):
                for m in range(CTX_IDX_MINOR // L):
                    vv = idx_c[i, pl.ds(m * L, L)]
                    idxp_c[i, pl.ds(m * L, L)] = (
                        ((vv >> 1) & ~PAIR_M) | (vv & PAIR_M))
                    hv_c[pl.ds(i * CTX_IDX_MINOR + m * L, L)] = (vv >> 8) & EMBED
            cps = []
            for i in range(C * CTX // CTX_IDX_MINOR):
                cps.append(pltpu.async_copy(
                    win_hbm.at[idxp_c.at[i]],
                    ctx_v.at[pl.ds(i * CTX_IDX_MINOR, CTX_IDX_MINOR)], sem))
            for cp in cps:
                cp.wait()

            def row_body(r, rc):
                base_c = r * CTX
                ro = pl.multiple_of(r * CTX, 4)
                hv1 = hv_c[pl.ds(ro, L)]
                hv2 = hv_c[pl.ds(ro + 4, L)]
                acc = [jnp.zeros((L,), jnp.float32) for _ in range(NSEG)]
                for c in range(CTX):
                    h = hv1[c] if c < L else hv2[c - 4]
                    for j in range(NSEG):
                        acc[j] = acc[j] + ctx_v[base_c + c, pl.ds(h + j * L, L)]
                so = pl.multiple_of((g * C + r) * EMBED, 16)
                for j in range(NSEG):
                    csum_v[pl.ds(so + j * L, L)] = acc[j]
                return rc

            lax.fori_loop(0, C, row_body, 0)
            return carry

        lax.fori_loop(0, NCHUNK, chunk_body, 0)
        pltpu.sync_copy(
            csum_v,
            out_hbm.at[pl.ds(wid * (ROWS_PER_W * EMBED), ROWS_PER_W * EMBED)])

    return k


def _sc_logits_kernel():
    @functools.partial(
        pl.kernel,
        out_type=jax.ShapeDtypeStruct((B * T,), jnp.float32),
        mesh=_mesh(),
        scratch_types=[
            pltpu.VMEM((C * T // TGT_IDX_MINOR, TGT_IDX_MINOR), jnp.int32),
            pltpu.VMEM((C * T // TGT_IDX_MINOR, TGT_IDX_MINOR), jnp.int32),
            pltpu.VMEM((C * 8 + 8,), jnp.int32),
            pltpu.VMEM((C * T, ROW_W), jnp.float32),
            pltpu.VMEM((C * EMBED,), jnp.float32),
            pltpu.VMEM((C * T * L,), jnp.float32),
            pltpu.VMEM((ROWS_PER_W * T,), jnp.float32),
            pltpu.SemaphoreType.DMA,
        ],
        compiler_params=pltpu.CompilerParams(needs_layout_passes=False,
                                             use_tc_tiling_on_sc=False),
    )
    def k(tgt_idx_hbm, wout_hbm, csum_hbm, out_hbm,
          idx_t, idxp_t, hv_t8, tgt_v, csum_v, part_v, log_v, sem):
        wid = lax.axis_index("s") * NC + lax.axis_index("c")
        tgt_off0 = wid * (ROWS_PER_W * T)
        csum_off0 = wid * (ROWS_PER_W * EMBED)
        lane = lax.iota(jnp.int32, L)

        def chunk_body(g, carry):
            t_off = pl.multiple_of(tgt_off0 + g * (C * T), 8)
            for i in range(C * T // TGT_IDX_MINOR):
                pltpu.sync_copy(
                    tgt_idx_hbm.at[pl.ds(t_off + i * TGT_IDX_MINOR,
                                         TGT_IDX_MINOR)],
                    idx_t.at[i])
            pltpu.sync_copy(
                csum_hbm.at[pl.ds(
                    pl.multiple_of(csum_off0 + g * (C * EMBED), 8),
                    C * EMBED)],
                csum_v)
            for i in range(C * T // TGT_IDX_MINOR):
                for m in range(TGT_IDX_MINOR // L):
                    vv = idx_t[i, pl.ds(m * L, L)]
                    idxp_t[i, pl.ds(m * L, L)] = (
                        ((vv >> 1) & ~PAIR_M) | (vv & PAIR_M))
                    tr_vec = (i * TGT_IDX_MINOR + m * L) + lane
                    pos = (tr_vec // T) * 8 + tr_vec % T
                    plsc.store_scatter(hv_t8, [pos], (vv >> 8) & EMBED)
            cps = []
            for i in range(C * T // TGT_IDX_MINOR):
                cps.append(pltpu.async_copy(
                    wout_hbm.at[idxp_t.at[i]],
                    tgt_v.at[pl.ds(i * TGT_IDX_MINOR, TGT_IDX_MINOR)], sem))
            for cp in cps:
                cp.wait()

            def row_body(r, rc):
                so = pl.multiple_of(r * EMBED, 16)
                acc = [csum_v[pl.ds(so + j * L, L)] for j in range(NSEG)]
                htv = hv_t8[pl.ds(pl.multiple_of(r * 8, 8), L)]
                for t in range(T):
                    tr = r * T + t
                    ht = htv[t]
                    s = acc[0] * tgt_v[tr, pl.ds(ht, L)]
                    for j in range(1, NSEG):
                        s = s + acc[j] * tgt_v[tr, pl.ds(ht + j * L, L)]
                    po = pl.multiple_of(tr * L, 16)
                    part_v[pl.ds(po, L)] = s
                return rc

            lax.fori_loop(0, C, row_body, 0)

            for m in range(C * T // L):
                idx0 = lane * L + (m * L * L)
                red = plsc.load_gather(part_v, [idx0])
                for kk in range(1, L):
                    red = red + plsc.load_gather(part_v, [idx0 + kk])
                lo = pl.multiple_of(g * (C * T) + m * L, 16)
                log_v[pl.ds(lo, L)] = red * (1.0 / CTX)
            return carry

        lax.fori_loop(0, NCHUNK, chunk_body, 0)
        pltpu.sync_copy(
            log_v,
            out_hbm.at[pl.ds(wid * (ROWS_PER_W * T), ROWS_PER_W * T)])

    return k


def _bce_kernel(logits_ref, labels_ref, out_ref):
    l = logits_ref[:, :]
    y = labels_ref[:, :]
    bce = jnp.maximum(l, 0.0) - l * y + jnp.log(1.0 + jnp.exp(-jnp.abs(l)))
    out_ref[0, 0] = jnp.sum(bce) * (1.0 / (B * T))


@jax.jit
def kernel(contexts, targets, labels, W_in, W_out):
    ctx_idx = contexts.astype(jnp.int32).reshape(B * CTX)
    tgt_idx = targets.astype(jnp.int32).reshape(B * T)
    win_c = _convert(W_in.T)
    csum = _sc_ctxsum_kernel()(ctx_idx, win_c)
    wout_c = _convert(W_out.T)
    logits = _sc_logits_kernel()(tgt_idx, wout_c, csum)

    labels_f = labels.astype(jnp.float32).reshape(B * T)
    loss2d = pl.pallas_call(
        _bce_kernel,
        out_shape=jax.ShapeDtypeStruct((1, 1), jnp.float32),
        in_specs=[pl.BlockSpec(memory_space=pltpu.VMEM),
                  pl.BlockSpec(memory_space=pltpu.VMEM)],
        out_specs=pl.BlockSpec(memory_space=pltpu.SMEM),
    )(logits.reshape(B * T // 128, 128),
      labels_f.reshape(B * T // 128, 128))
    return loss2d[0, 0]


# speedup vs baseline: 1.4971x; 1.0681x over previous
import functools

import jax
import jax.numpy as jnp
from jax import lax
from jax.experimental import pallas as pl
from jax.experimental.pallas import tpu as pltpu
from jax.experimental.pallas import tpu_sc as plsc

VOCAB = 1000000
EMBED = 64
B = 16384
CTX = 20
T = 5

NC, NS, L = 2, 16, 16
NW = NC * NS
ROWS_PER_W = B // NW
C = 32
NCHUNK = ROWS_PER_W // C
NSEG = EMBED // L
ROW_W = 128
CTX_IDX_MINOR = 128
TGT_IDX_MINOR = 80

VB = 32768
VBH = VB // 2
PAIR_M = VBH - 1
CONV_GRID = (VOCAB + VB - 1) // VB
CONV_ROWS = CONV_GRID * VBH


def _transpose_kernel(wt_ref, out_ref):
    t = jnp.transpose(wt_ref[:, :])
    out_ref[:, 0:EMBED] = t[0:VBH]
    out_ref[:, EMBED:ROW_W] = t[VBH:VB]


def _convert(wt):
    return pl.pallas_call(
        _transpose_kernel,
        grid=(CONV_GRID,),
        in_specs=[pl.BlockSpec((EMBED, VB), lambda g: (0, g))],
        out_specs=pl.BlockSpec((VBH, ROW_W), lambda g: (g, 0)),
        out_shape=jax.ShapeDtypeStruct((CONV_ROWS, ROW_W), jnp.float32),
    )(wt).reshape(CONV_ROWS * 2, EMBED)


_MESH = None


def _mesh():
    global _MESH
    if _MESH is None:
        _MESH = plsc.VectorSubcoreMesh(
            core_axis_name="c", subcore_axis_name="s",
            num_cores=NC, num_subcores=NS)
    return _MESH


def _sc_ctxsum_kernel():
    @functools.partial(
        pl.kernel,
        out_type=jax.ShapeDtypeStruct((B * EMBED,), jnp.float32),
        mesh=_mesh(),
        scratch_types=[
            pltpu.VMEM((C * CTX // CTX_IDX_MINOR, CTX_IDX_MINOR), jnp.int32),
            pltpu.VMEM((C * CTX // CTX_IDX_MINOR, CTX_IDX_MINOR), jnp.int32),
            pltpu.VMEM((C * CTX, EMBED), jnp.float32),
            pltpu.VMEM((ROWS_PER_W * EMBED,), jnp.float32),
            pltpu.SemaphoreType.DMA,
        ],
        compiler_params=pltpu.CompilerParams(needs_layout_passes=False,
                                             use_tc_tiling_on_sc=False),
    )
    def k(ctx_idx_hbm, win_hbm, out_hbm, idx_c, idxp_c, ctx_v,
          csum_v, sem):
        wid = lax.axis_index("s") * NC + lax.axis_index("c")
        ctx_off0 = wid * (ROWS_PER_W * CTX)

        def chunk_body(g, carry):
            c_off = pl.multiple_of(ctx_off0 + g * (C * CTX), 8)
            for i in range(C * CTX // CTX_IDX_MINOR):
                pltpu.sync_copy(
                    ctx_idx_hbm.at[pl.ds(c_off + i * CTX_IDX_MINOR,
                                         CTX_IDX_MINOR)],
                    idx_c.at[i])
            for i in range(C * CTX // CTX_IDX_MINOR):
                for m in range(CTX_IDX_MINOR // L):
                    vv = idx_c[i, pl.ds(m * L, L)]
                    t = ((vv >> 1) & ~PAIR_M) | (vv & PAIR_M)
                    idxp_c[i, pl.ds(m * L, L)] = (t << 1) | ((vv >> 14) & 1)
            cps = []
            for i in range(C * CTX // CTX_IDX_MINOR):
                cps.append(pltpu.async_copy(
                    win_hbm.at[idxp_c.at[i]],
                    ctx_v.at[pl.ds(i * CTX_IDX_MINOR, CTX_IDX_MINOR)], sem))
            for cp in cps:
                cp.wait()

            def row_body(r, rc):
                base_c = r * CTX
                acc = [jnp.zeros((L,), jnp.float32) for _ in range(NSEG)]
                for c in range(CTX):
                    for j in range(NSEG):
                        acc[j] = acc[j] + ctx_v[base_c + c, pl.ds(j * L, L)]
                so = pl.multiple_of((g * C + r) * EMBED, 16)
                for j in range(NSEG):
                    csum_v[pl.ds(so + j * L, L)] = acc[j]
                return rc

            lax.fori_loop(0, C, row_body, 0)
            return carry

        lax.fori_loop(0, NCHUNK, chunk_body, 0)
        pltpu.sync_copy(
            csum_v,
            out_hbm.at[pl.ds(wid * (ROWS_PER_W * EMBED), ROWS_PER_W * EMBED)])

    return k


def _sc_logits_kernel():
    @functools.partial(
        pl.kernel,
        out_type=jax.ShapeDtypeStruct((B * T,), jnp.float32),
        mesh=_mesh(),
        scratch_types=[
            pltpu.VMEM((C * T // TGT_IDX_MINOR, TGT_IDX_MINOR), jnp.int32),
            pltpu.VMEM((C * T // TGT_IDX_MINOR, TGT_IDX_MINOR), jnp.int32),
            pltpu.VMEM((C * T, EMBED), jnp.float32),
            pltpu.VMEM((C * EMBED,), jnp.float32),
            pltpu.VMEM((C * T * L,), jnp.float32),
            pltpu.VMEM((ROWS_PER_W * T,), jnp.float32),
            pltpu.SemaphoreType.DMA,
        ],
        compiler_params=pltpu.CompilerParams(needs_layout_passes=False,
                                             use_tc_tiling_on_sc=False),
    )
    def k(tgt_idx_hbm, wout_hbm, csum_hbm, out_hbm,
          idx_t, idxp_t, tgt_v, csum_v, part_v, log_v, sem):
        wid = lax.axis_index("s") * NC + lax.axis_index("c")
        tgt_off0 = wid * (ROWS_PER_W * T)
        csum_off0 = wid * (ROWS_PER_W * EMBED)
        lane = lax.iota(jnp.int32, L)

        def chunk_body(g, carry):
            t_off = pl.multiple_of(tgt_off0 + g * (C * T), 8)
            for i in range(C * T // TGT_IDX_MINOR):
                pltpu.sync_copy(
                    tgt_idx_hbm.at[pl.ds(t_off + i * TGT_IDX_MINOR,
                                         TGT_IDX_MINOR)],
                    idx_t.at[i])
            pltpu.sync_copy(
                csum_hbm.at[pl.ds(
                    pl.multiple_of(csum_off0 + g * (C * EMBED), 8),
                    C * EMBED)],
                csum_v)
            for i in range(C * T // TGT_IDX_MINOR):
                for m in range(TGT_IDX_MINOR // L):
                    vv = idx_t[i, pl.ds(m * L, L)]
                    t = ((vv >> 1) & ~PAIR_M) | (vv & PAIR_M)
                    idxp_t[i, pl.ds(m * L, L)] = (t << 1) | ((vv >> 14) & 1)
            cps = []
            for i in range(C * T // TGT_IDX_MINOR):
                cps.append(pltpu.async_copy(
                    wout_hbm.at[idxp_t.at[i]],
                    tgt_v.at[pl.ds(i * TGT_IDX_MINOR, TGT_IDX_MINOR)], sem))
            for cp in cps:
                cp.wait()

            def row_body(r, rc):
                so = pl.multiple_of(r * EMBED, 16)
                acc = [csum_v[pl.ds(so + j * L, L)] for j in range(NSEG)]
                for t in range(T):
                    tr = r * T + t
                    s = acc[0] * tgt_v[tr, pl.ds(0, L)]
                    for j in range(1, NSEG):
                        s = s + acc[j] * tgt_v[tr, pl.ds(j * L, L)]
                    po = pl.multiple_of(tr * L, 16)
                    part_v[pl.ds(po, L)] = s
                return rc

            lax.fori_loop(0, C, row_body, 0)

            for m in range(C * T // L):
                idx0 = lane * L + (m * L * L)
                red = plsc.load_gather(part_v, [idx0])
                for kk in range(1, L):
                    red = red + plsc.load_gather(part_v, [idx0 + kk])
                lo = pl.multiple_of(g * (C * T) + m * L, 16)
                log_v[pl.ds(lo, L)] = red * (1.0 / CTX)
            return carry

        lax.fori_loop(0, NCHUNK, chunk_body, 0)
        pltpu.sync_copy(
            log_v,
            out_hbm.at[pl.ds(wid * (ROWS_PER_W * T), ROWS_PER_W * T)])

    return k


def _bce_kernel(logits_ref, labels_ref, out_ref):
    l = logits_ref[:, :]
    y = labels_ref[:, :]
    bce = jnp.maximum(l, 0.0) - l * y + jnp.log(1.0 + jnp.exp(-jnp.abs(l)))
    out_ref[0, 0] = jnp.sum(bce) * (1.0 / (B * T))


@jax.jit
def kernel(contexts, targets, labels, W_in, W_out):
    ctx_idx = contexts.astype(jnp.int32).reshape(B * CTX)
    tgt_idx = targets.astype(jnp.int32).reshape(B * T)
    win_c = _convert(W_in.T)
    csum = _sc_ctxsum_kernel()(ctx_idx, win_c)
    wout_c = _convert(W_out.T)
    logits = _sc_logits_kernel()(tgt_idx, wout_c, csum)

    labels_f = labels.astype(jnp.float32).reshape(B * T)
    loss2d = pl.pallas_call(
        _bce_kernel,
        out_shape=jax.ShapeDtypeStruct((1, 1), jnp.float32),
        in_specs=[pl.BlockSpec(memory_space=pltpu.VMEM),
                  pl.BlockSpec(memory_space=pltpu.VMEM)],
        out_specs=pl.BlockSpec(memory_space=pltpu.SMEM),
    )(logits.reshape(B * T // 128, 128),
      labels_f.reshape(B * T // 128, 128))
    return loss2d[0, 0]


# speedup vs baseline: 1.5055x; 1.0056x over previous
import functools

import jax
import jax.numpy as jnp
from jax import lax
from jax.experimental import pallas as pl
from jax.experimental.pallas import tpu as pltpu
from jax.experimental.pallas import tpu_sc as plsc

VOCAB = 1000000
EMBED = 64
B = 16384
CTX = 20
T = 5

NC, NS, L = 2, 16, 16
NW = NC * NS
ROWS_PER_W = B // NW
C = 64
NCHUNK = ROWS_PER_W // C
NSEG = EMBED // L
ROW_W = 128
CTX_IDX_MINOR = 128
TGT_IDX_MINOR = 80

VB = 32768
VBH = VB // 2
PAIR_M = VBH - 1
CONV_GRID = (VOCAB + VB - 1) // VB
CONV_ROWS = CONV_GRID * VBH


def _transpose_kernel(wt_ref, out_ref):
    t = jnp.transpose(wt_ref[:, :])
    out_ref[:, 0:EMBED] = t[0:VBH]
    out_ref[:, EMBED:ROW_W] = t[VBH:VB]


def _convert(wt):
    return pl.pallas_call(
        _transpose_kernel,
        grid=(CONV_GRID,),
        in_specs=[pl.BlockSpec((EMBED, VB), lambda g: (0, g))],
        out_specs=pl.BlockSpec((VBH, ROW_W), lambda g: (g, 0)),
        out_shape=jax.ShapeDtypeStruct((CONV_ROWS, ROW_W), jnp.float32),
    )(wt).reshape(CONV_ROWS * 2, EMBED)


_MESH = None


def _mesh():
    global _MESH
    if _MESH is None:
        _MESH = plsc.VectorSubcoreMesh(
            core_axis_name="c", subcore_axis_name="s",
            num_cores=NC, num_subcores=NS)
    return _MESH


def _sc_ctxsum_kernel():
    @functools.partial(
        pl.kernel,
        out_type=jax.ShapeDtypeStruct((B * EMBED,), jnp.float32),
        mesh=_mesh(),
        scratch_types=[
            pltpu.VMEM((C * CTX // CTX_IDX_MINOR, CTX_IDX_MINOR), jnp.int32),
            pltpu.VMEM((C * CTX // CTX_IDX_MINOR, CTX_IDX_MINOR), jnp.int32),
            pltpu.VMEM((C * CTX, EMBED), jnp.float32),
            pltpu.VMEM((ROWS_PER_W * EMBED,), jnp.float32),
            pltpu.SemaphoreType.DMA,
        ],
        compiler_params=pltpu.CompilerParams(needs_layout_passes=False,
                                             use_tc_tiling_on_sc=False),
    )
    def k(ctx_idx_hbm, win_hbm, out_hbm, idx_c, idxp_c, ctx_v,
          csum_v, sem):
        wid = lax.axis_index("s") * NC + lax.axis_index("c")
        ctx_off0 = wid * (ROWS_PER_W * CTX)

        def chunk_body(g, carry):
            c_off = pl.multiple_of(ctx_off0 + g * (C * CTX), 8)
            for i in range(C * CTX // CTX_IDX_MINOR):
                pltpu.sync_copy(
                    ctx_idx_hbm.at[pl.ds(c_off + i * CTX_IDX_MINOR,
                                         CTX_IDX_MINOR)],
                    idx_c.at[i])
            for i in range(C * CTX // CTX_IDX_MINOR):
                for m in range(CTX_IDX_MINOR // L):
                    vv = idx_c[i, pl.ds(m * L, L)]
                    t = ((vv >> 1) & ~PAIR_M) | (vv & PAIR_M)
                    idxp_c[i, pl.ds(m * L, L)] = (t << 1) | ((vv >> 14) & 1)
            cps = []
            for i in range(C * CTX // CTX_IDX_MINOR):
                cps.append(pltpu.async_copy(
                    win_hbm.at[idxp_c.at[i]],
                    ctx_v.at[pl.ds(i * CTX_IDX_MINOR, CTX_IDX_MINOR)], sem))
            for cp in cps:
                cp.wait()

            def row_body(r, rc):
                base_c = r * CTX
                acc = [jnp.zeros((L,), jnp.float32) for _ in range(NSEG)]
                for c in range(CTX):
                    for j in range(NSEG):
                        acc[j] = acc[j] + ctx_v[base_c + c, pl.ds(j * L, L)]
                so = pl.multiple_of((g * C + r) * EMBED, 16)
                for j in range(NSEG):
                    csum_v[pl.ds(so + j * L, L)] = acc[j]
                return rc

            lax.fori_loop(0, C, row_body, 0)
            return carry

        lax.fori_loop(0, NCHUNK, chunk_body, 0)
        pltpu.sync_copy(
            csum_v,
            out_hbm.at[pl.ds(wid * (ROWS_PER_W * EMBED), ROWS_PER_W * EMBED)])

    return k


def _sc_logits_kernel():
    @functools.partial(
        pl.kernel,
        out_type=jax.ShapeDtypeStruct((B * T,), jnp.float32),
        mesh=_mesh(),
        scratch_types=[
            pltpu.VMEM((C * T // TGT_IDX_MINOR, TGT_IDX_MINOR), jnp.int32),
            pltpu.VMEM((C * T // TGT_IDX_MINOR, TGT_IDX_MINOR), jnp.int32),
            pltpu.VMEM((C * T, EMBED), jnp.float32),
            pltpu.VMEM((C * EMBED,), jnp.float32),
            pltpu.VMEM((C * T * L,), jnp.float32),
            pltpu.VMEM((ROWS_PER_W * T,), jnp.float32),
            pltpu.SemaphoreType.DMA,
        ],
        compiler_params=pltpu.CompilerParams(needs_layout_passes=False,
                                             use_tc_tiling_on_sc=False),
    )
    def k(tgt_idx_hbm, wout_hbm, csum_hbm, out_hbm,
          idx_t, idxp_t, tgt_v, csum_v, part_v, log_v, sem):
        wid = lax.axis_index("s") * NC + lax.axis_index("c")
        tgt_off0 = wid * (ROWS_PER_W * T)
        csum_off0 = wid * (ROWS_PER_W * EMBED)
        lane = lax.iota(jnp.int32, L)

        def chunk_body(g, carry):
            t_off = pl.multiple_of(tgt_off0 + g * (C * T), 8)
            for i in range(C * T // TGT_IDX_MINOR):
                pltpu.sync_copy(
                    tgt_idx_hbm.at[pl.ds(t_off + i * TGT_IDX_MINOR,
                                         TGT_IDX_MINOR)],
                    idx_t.at[i])
            pltpu.sync_copy(
                csum_hbm.at[pl.ds(
                    pl.multiple_of(csum_off0 + g * (C * EMBED), 8),
                    C * EMBED)],
                csum_v)
            for i in range(C * T // TGT_IDX_MINOR):
                for m in range(TGT_IDX_MINOR // L):
                    vv = idx_t[i, pl.ds(m * L, L)]
                    t = ((vv >> 1) & ~PAIR_M) | (vv & PAIR_M)
                    idxp_t[i, pl.ds(m * L, L)] = (t << 1) | ((vv >> 14) & 1)
            cps = []
            for i in range(C * T // TGT_IDX_MINOR):
                cps.append(pltpu.async_copy(
                    wout_hbm.at[idxp_t.at[i]],
                    tgt_v.at[pl.ds(i * TGT_IDX_MINOR, TGT_IDX_MINOR)], sem))
            for cp in cps:
                cp.wait()

            def row_body(r, rc):
                so = pl.multiple_of(r * EMBED, 16)
                acc = [csum_v[pl.ds(so + j * L, L)] for j in range(NSEG)]
                for t in range(T):
                    tr = r * T + t
                    s = acc[0] * tgt_v[tr, pl.ds(0, L)]
                    for j in range(1, NSEG):
                        s = s + acc[j] * tgt_v[tr, pl.ds(j * L, L)]
                    po = pl.multiple_of(tr * L, 16)
                    part_v[pl.ds(po, L)] = s
                return rc

            lax.fori_loop(0, C, row_body, 0)

            for m in range(C * T // L):
                idx0 = lane * L + (m * L * L)
                red = plsc.load_gather(part_v, [idx0])
                for kk in range(1, L):
                    red = red + plsc.load_gather(part_v, [idx0 + kk])
                lo = pl.multiple_of(g * (C * T) + m * L, 16)
                log_v[pl.ds(lo, L)] = red * (1.0 / CTX)
            return carry

        lax.fori_loop(0, NCHUNK, chunk_body, 0)
        pltpu.sync_copy(
            log_v,
            out_hbm.at[pl.ds(wid * (ROWS_PER_W * T), ROWS_PER_W * T)])

    return k


def _bce_kernel(logits_ref, labels_ref, out_ref):
    l = logits_ref[:, :]
    y = labels_ref[:, :]
    bce = jnp.maximum(l, 0.0) - l * y + jnp.log(1.0 + jnp.exp(-jnp.abs(l)))
    out_ref[0, 0] = jnp.sum(bce) * (1.0 / (B * T))


@jax.jit
def kernel(contexts, targets, labels, W_in, W_out):
    ctx_idx = contexts.astype(jnp.int32).reshape(B * CTX)
    tgt_idx = targets.astype(jnp.int32).reshape(B * T)
    win_c = _convert(W_in.T)
    csum = _sc_ctxsum_kernel()(ctx_idx, win_c)
    wout_c = _convert(W_out.T)
    logits = _sc_logits_kernel()(tgt_idx, wout_c, csum)

    labels_f = labels.astype(jnp.float32).reshape(B * T)
    loss2d = pl.pallas_call(
        _bce_kernel,
        out_shape=jax.ShapeDtypeStruct((1, 1), jnp.float32),
        in_specs=[pl.BlockSpec(memory_space=pltpu.VMEM),
                  pl.BlockSpec(memory_space=pltpu.VMEM)],
        out_specs=pl.BlockSpec(memory_space=pltpu.SMEM),
    )(logits.reshape(B * T // 128, 128),
      labels_f.reshape(B * T // 128, 128))
    return loss2d[0, 0]
